# baseline (speedup 1.0000x reference)
import functools

import numpy as np

import jax
import jax.numpy as jnp
from jax import lax
from jax.experimental import pallas as pl
from jax.experimental.pallas import tpu as pltpu
from jax.experimental.pallas import tpu_sc as plsc

DIM = 128
CHUNK = 64


def _stage_perm() -> np.ndarray:
    c = np.arange(DIM // 2, dtype=np.int32)
    p_low = 32 * (c // 16) + (c % 16)
    return np.concatenate([p_low, p_low + 16])


@functools.lru_cache(maxsize=None)
def _make_gather(B: int, E: int, R: int):
    info = plsc.get_sparse_core_info()
    NC, NS = info.num_cores, info.num_subcores
    NW = NC * NS
    assert B % (8 * NW) == 0
    b_per_w = B // NW
    assert b_per_w % CHUNK == 0
    n_chunks = b_per_w // CHUNK

    mesh = plsc.VectorSubcoreMesh(core_axis_name="c", subcore_axis_name="s")

    @functools.partial(
        pl.kernel,
        mesh=mesh,
        out_type=(
            jax.ShapeDtypeStruct((B, DIM // 2), jnp.int32),
            jax.ShapeDtypeStruct((B, DIM // 2), jnp.int32),
        ),
        scratch_types=[
            pltpu.VMEM((CHUNK,), jnp.int32),
            pltpu.VMEM((CHUNK,), jnp.int32),
            pltpu.VMEM((CHUNK,), jnp.int32),
            pltpu.VMEM((CHUNK,), jnp.int32),
            pltpu.VMEM((CHUNK,), jnp.int32),
            pltpu.VMEM((CHUNK,), jnp.int32),
            pltpu.VMEM((CHUNK, DIM), jnp.float32),
            pltpu.VMEM((CHUNK, DIM), jnp.float32),
            pltpu.VMEM((CHUNK, DIM), jnp.float32),
            pltpu.VMEM((CHUNK, DIM), jnp.float32),
            pltpu.VMEM((CHUNK, DIM), jnp.float32),
            pltpu.VMEM((CHUNK, DIM), jnp.float32),
            pltpu.VMEM((CHUNK, DIM // 2), jnp.int32),
            pltpu.VMEM((CHUNK, DIM // 2), jnp.int32),
            pltpu.VMEM((CHUNK, DIM // 2), jnp.int32),
            pltpu.VMEM((CHUNK, DIM // 2), jnp.int32),
            pltpu.SemaphoreType.DMA,
            pltpu.SemaphoreType.DMA,
            pltpu.SemaphoreType.DMA,
            pltpu.SemaphoreType.DMA,
            pltpu.SemaphoreType.DMA,
            pltpu.SemaphoreType.DMA,
            pltpu.SemaphoreType.DMA,
            pltpu.SemaphoreType.DMA,
        ],
    )
    def gather_k(ent_hbm, rel_hbm, hidx_hbm, tidx_hbm, ridx_hbm,
                 h_out, p_out,
                 ih0, ih1, it0, it1, ir0, ir1,
                 bh0, bh1, bt0, bt1, br0, br1,
                 oh0, oh1, op0, op1,
                 gsh0, gsh1, gst0, gst1, gsr0, gsr1, wsh, wsp):
        wid = lax.axis_index("s") * NC + lax.axis_index("c")
        base = wid * b_per_w
        ih = (ih0, ih1)
        it = (it0, it1)
        ir = (ir0, ir1)
        bh = (bh0, bh1)
        bt = (bt0, bt1)
        br = (br0, br1)
        oh = (oh0, oh1)
        op = (op0, op1)
        gsh = (gsh0, gsh1)
        gst = (gst0, gst1)
        gsr = (gsr0, gsr1)

        c7fff = jnp.full((16,), 0x7FFF, jnp.int32)
        c1 = jnp.full((16,), 1, jnp.int32)
        c16 = jnp.full((16,), 16, jnp.int32)

        def bf16_bits(v):
            bits = lax.bitcast_convert_type(v, jnp.int32)
            rnd = bits + c7fff + (lax.shift_right_logical(bits, c16) & c1)
            return lax.shift_right_logical(rnd, c16)

        def pack_pair(a, b):
            return bf16_bits(a) | lax.shift_left(bf16_bits(b), c16)

        def pack_chunk(src, dst):
            def body(row, _):
                for g in range(DIM // 32):
                    a = src[row, pl.ds(32 * g, 16)]
                    b = src[row, pl.ds(32 * g + 16, 16)]
                    dst[row, pl.ds(16 * g, 16)] = pack_pair(a, b)
                return 0
            lax.fori_loop(0, CHUNK, body, 0)

        def mulpack_chunk(ts, rs, dst):
            def body(row, _):
                for g in range(DIM // 32):
                    a = (ts[row, pl.ds(32 * g, 16)]
                         * rs[row, pl.ds(32 * g, 16)])
                    b = (ts[row, pl.ds(32 * g + 16, 16)]
                         * rs[row, pl.ds(32 * g + 16, 16)])
                    dst[row, pl.ds(16 * g, 16)] = pack_pair(a, b)
                return 0
            lax.fori_loop(0, CHUNK, body, 0)

        def start_unit(c):
            b = c & 1
            off = base + c * CHUNK
            pltpu.sync_copy(hidx_hbm.at[pl.ds(off, CHUNK)], ih[b])
            pltpu.sync_copy(tidx_hbm.at[pl.ds(off, CHUNK)], it[b])
            pltpu.sync_copy(ridx_hbm.at[pl.ds(off, CHUNK)], ir[b])
            return (pltpu.async_copy(ent_hbm.at[ih[b]], bh[b], gsh[b]),
                    pltpu.async_copy(ent_hbm.at[it[b]], bt[b], gst[b]),
                    pltpu.async_copy(rel_hbm.at[ir[b]], br[b], gsr[b]))

        def finish_unit(c, handles):
            b = c & 1
            off = base + c * CHUNK
            hh, ht, hr = handles
            hh.wait()
            pack_chunk(bh[b], oh[b])
            wh = pltpu.async_copy(oh[b], h_out.at[pl.ds(off, CHUNK)], wsh)
            ht.wait()
            hr.wait()
            mulpack_chunk(bt[b], br[b], op[b])
            wp = pltpu.async_copy(op[b], p_out.at[pl.ds(off, CHUNK)], wsp)
            return wh, wp

        pend = [None] * n_chunks
        writes = [None] * n_chunks
        for c in range(n_chunks):
            if c >= 2:
                for w in writes[c - 2]:
                    w.wait()
            pend[c] = start_unit(c)
            if c >= 1:
                writes[c - 1] = finish_unit(c - 1, pend[c - 1])
        writes[n_chunks - 1] = finish_unit(n_chunks - 1, pend[n_chunks - 1])
        if n_chunks >= 2:
            for w in writes[n_chunks - 2]:
                w.wait()
        for w in writes[n_chunks - 1]:
            w.wait()

    return gather_k


def _mlp_body(h_ref, p_ref, W1_ref, b1_ref, W2_ref, b2_ref,
              Wp_ref, bp_ref, out_ref):
    wh = h_ref[...]
    wp = p_ref[...]
    hi_mask = jnp.int32(-65536)
    x = jnp.concatenate([
        lax.bitcast_convert_type(lax.shift_left(wh, 16), jnp.float32),
        lax.bitcast_convert_type(wh & hi_mask, jnp.float32),
        lax.bitcast_convert_type(lax.shift_left(wp, 16), jnp.float32),
        lax.bitcast_convert_type(wp & hi_mask, jnp.float32),
    ], axis=1)
    ss = jnp.sum(x * x, axis=1, keepdims=True)
    inv = 1.0 / jnp.maximum(jnp.sqrt(ss), 1e-12)
    x = (x * inv).astype(jnp.bfloat16)
    y = lax.dot_general(x, W1_ref[...], (((1,), (0,)), ((), ())),
                        preferred_element_type=jnp.float32) + b1_ref[...]
    y = jnp.maximum(y, 0.0).astype(jnp.bfloat16)
    y = lax.dot_general(y, W2_ref[...], (((1,), (0,)), ((), ())),
                        preferred_element_type=jnp.float32) + b2_ref[...]
    y = jnp.maximum(y, 0.0)
    s = jnp.sum(y * Wp_ref[...], axis=1, keepdims=True) + bp_ref[...]
    out_ref[...] = jax.nn.sigmoid(s)


@functools.lru_cache(maxsize=None)
def _make_mlp(H: int, blk: int):
    grid = (H // blk,)
    full = lambda i: (0, 0)
    return pl.pallas_call(
        _mlp_body,
        grid=grid,
        in_specs=[
            pl.BlockSpec((blk, DIM // 2), lambda i: (i, 0)),
            pl.BlockSpec((blk, DIM // 2), lambda i: (i, 0)),
            pl.BlockSpec((2 * DIM, 512), full),
            pl.BlockSpec((1, 512), full),
            pl.BlockSpec((512, 256), full),
            pl.BlockSpec((1, 256), full),
            pl.BlockSpec((1, 256), full),
            pl.BlockSpec((1, 1), full),
        ],
        out_specs=pl.BlockSpec((blk, 1), lambda i: (i, 0)),
        out_shape=jax.ShapeDtypeStruct((H, 1), jnp.float32),
    )


_PERM = _stage_perm()
_PERM256 = np.concatenate([_PERM, _PERM + DIM])


def kernel(data, eval, cf_train, ent_emb, rel_emb, W1, b1, W2, b2, Wp, bp):
    B = data.shape[0]
    hidx = data[:, 0]
    tidx = data[:, 1]
    ridx = data[:, 2]
    W1b = W1.T[_PERM256].astype(jnp.bfloat16)
    W2b = W2.T.astype(jnp.bfloat16)
    b1r = b1.reshape(1, -1)
    b2r = b2.reshape(1, -1)
    bpr = bp.reshape(1, 1)
    H = B // 2
    gather = _make_gather(H, ent_emb.shape[0], rel_emb.shape[0])
    mlp = _make_mlp(H, 4096)
    scores = []
    for lo in (0, H):
        h, p = gather(ent_emb, rel_emb,
                      lax.slice(hidx, (lo,), (lo + H,)),
                      lax.slice(tidx, (lo,), (lo + H,)),
                      lax.slice(ridx, (lo,), (lo + H,)))
        scores.append(mlp(h, p, W1b, b1r, W2b, b2r, Wp, bpr))
    return jnp.concatenate(scores, axis=0)


# speedup vs baseline: 1.1768x; 1.1768x over previous
import functools

import jax
import jax.numpy as jnp
from jax import lax
from jax.experimental import pallas as pl
from jax.experimental.pallas import tpu as pltpu
from jax.experimental.pallas import tpu_sc as plsc

DIM = 128
CHUNK = 128
NSPLIT = 4
MLP_BLK = 2048


@functools.lru_cache(maxsize=None)
def _make_gather(B: int, E: int, R: int):
    info = plsc.get_sparse_core_info()
    NC, NS = info.num_cores, info.num_subcores
    NW = NC * NS
    assert B % (8 * NW) == 0
    b_per_w = B // NW
    assert b_per_w % CHUNK == 0
    n_chunks = b_per_w // CHUNK

    mesh = plsc.VectorSubcoreMesh(core_axis_name="c", subcore_axis_name="s")

    @functools.partial(
        pl.kernel,
        mesh=mesh,
        out_type=(
            jax.ShapeDtypeStruct((B, DIM), jnp.float32),
            jax.ShapeDtypeStruct((B, DIM), jnp.float32),
        ),
        scratch_types=[
            pltpu.VMEM((CHUNK,), jnp.int32),
            pltpu.VMEM((CHUNK,), jnp.int32),
            pltpu.VMEM((CHUNK,), jnp.int32),
            pltpu.VMEM((CHUNK,), jnp.int32),
            pltpu.VMEM((CHUNK,), jnp.int32),
            pltpu.VMEM((CHUNK,), jnp.int32),
            pltpu.VMEM((CHUNK, DIM), jnp.float32),
            pltpu.VMEM((CHUNK, DIM), jnp.float32),
            pltpu.VMEM((CHUNK, DIM), jnp.float32),
            pltpu.VMEM((CHUNK, DIM), jnp.float32),
            pltpu.VMEM((CHUNK, DIM), jnp.float32),
            pltpu.VMEM((CHUNK, DIM), jnp.float32),
            pltpu.SemaphoreType.DMA,
            pltpu.SemaphoreType.DMA,
            pltpu.SemaphoreType.DMA,
            pltpu.SemaphoreType.DMA,
            pltpu.SemaphoreType.DMA,
            pltpu.SemaphoreType.DMA,
            pltpu.SemaphoreType.DMA,
            pltpu.SemaphoreType.DMA,
        ],
    )
    def gather_k(ent_hbm, rel_hbm, hidx_hbm, tidx_hbm, ridx_hbm,
                 h_out, p_out,
                 ih0, ih1, it0, it1, ir0, ir1,
                 bh0, bh1, bt0, bt1, br0, br1,
                 gsh0, gsh1, gst0, gst1, gsr0, gsr1, wsh, wsp):
        wid = lax.axis_index("s") * NC + lax.axis_index("c")
        base = wid * b_per_w
        ih = (ih0, ih1)
        it = (it0, it1)
        ir = (ir0, ir1)
        bh = (bh0, bh1)
        bt = (bt0, bt1)
        br = (br0, br1)
        gsh = (gsh0, gsh1)
        gst = (gst0, gst1)
        gsr = (gsr0, gsr1)

        def mul_into(tb, rb):
            def body(row, _):
                for g in range(DIM // 16):
                    sl = pl.ds(g * 16, 16)
                    tb[row, sl] = tb[row, sl] * rb[row, sl]
                return 0
            lax.fori_loop(0, CHUNK, body, 0)

        def start_unit(c):
            b = c & 1
            off = base + c * CHUNK
            pltpu.sync_copy(hidx_hbm.at[pl.ds(off, CHUNK)], ih[b])
            pltpu.sync_copy(tidx_hbm.at[pl.ds(off, CHUNK)], it[b])
            pltpu.sync_copy(ridx_hbm.at[pl.ds(off, CHUNK)], ir[b])
            return (pltpu.async_copy(ent_hbm.at[ih[b]], bh[b], gsh[b]),
                    pltpu.async_copy(ent_hbm.at[it[b]], bt[b], gst[b]),
                    pltpu.async_copy(rel_hbm.at[ir[b]], br[b], gsr[b]))

        def finish_unit(c, handles):
            b = c & 1
            off = base + c * CHUNK
            hh, ht, hr = handles
            hh.wait()
            wh = pltpu.async_copy(bh[b], h_out.at[pl.ds(off, CHUNK)], wsh)
            ht.wait()
            hr.wait()
            mul_into(bt[b], br[b])
            wp = pltpu.async_copy(bt[b], p_out.at[pl.ds(off, CHUNK)], wsp)
            return wh, wp

        pend = [None] * n_chunks
        writes = [None] * n_chunks
        for c in range(n_chunks):
            if c >= 2:
                for w in writes[c - 2]:
                    w.wait()
            pend[c] = start_unit(c)
            if c >= 1:
                writes[c - 1] = finish_unit(c - 1, pend[c - 1])
        writes[n_chunks - 1] = finish_unit(n_chunks - 1, pend[n_chunks - 1])
        if n_chunks >= 2:
            for w in writes[n_chunks - 2]:
                w.wait()
        for w in writes[n_chunks - 1]:
            w.wait()

    return gather_k


def _mlp_body(h_ref, p_ref, W1_ref, b1_ref, W2_ref, b2_ref,
              Wp_ref, bp_ref, out_ref):
    x1 = h_ref[...]
    x2 = p_ref[...]
    ss = (jnp.sum(x1 * x1, axis=1, keepdims=True)
          + jnp.sum(x2 * x2, axis=1, keepdims=True))
    inv = 1.0 / jnp.maximum(jnp.sqrt(ss), 1e-12)
    x = jnp.concatenate([x1 * inv, x2 * inv], axis=1).astype(jnp.bfloat16)
    y = lax.dot_general(x, W1_ref[...], (((1,), (0,)), ((), ())),
                        preferred_element_type=jnp.float32) + b1_ref[...]
    y = jnp.maximum(y, 0.0).astype(jnp.bfloat16)
    y = lax.dot_general(y, W2_ref[...], (((1,), (0,)), ((), ())),
                        preferred_element_type=jnp.float32) + b2_ref[...]
    y = jnp.maximum(y, 0.0)
    s = jnp.sum(y * Wp_ref[...], axis=1, keepdims=True) + bp_ref[...]
    out_ref[...] = jax.nn.sigmoid(s)


@functools.lru_cache(maxsize=None)
def _make_mlp(H: int, blk: int):
    grid = (H // blk,)
    full = lambda i: (0, 0)
    return pl.pallas_call(
        _mlp_body,
        grid=grid,
        in_specs=[
            pl.BlockSpec((blk, DIM), lambda i: (i, 0)),
            pl.BlockSpec((blk, DIM), lambda i: (i, 0)),
            pl.BlockSpec((2 * DIM, 512), full),
            pl.BlockSpec((1, 512), full),
            pl.BlockSpec((512, 256), full),
            pl.BlockSpec((1, 256), full),
            pl.BlockSpec((1, 256), full),
            pl.BlockSpec((1, 1), full),
        ],
        out_specs=pl.BlockSpec((blk, 1), lambda i: (i, 0)),
        out_shape=jax.ShapeDtypeStruct((H, 1), jnp.float32),
    )


def kernel(data, eval, cf_train, ent_emb, rel_emb, W1, b1, W2, b2, Wp, bp):
    B = data.shape[0]
    hidx = data[:, 0]
    tidx = data[:, 1]
    ridx = data[:, 2]
    W1b = W1.T.astype(jnp.bfloat16)
    W2b = W2.T.astype(jnp.bfloat16)
    b1r = b1.reshape(1, -1)
    b2r = b2.reshape(1, -1)
    bpr = bp.reshape(1, 1)
    H = B // NSPLIT
    gather = _make_gather(H, ent_emb.shape[0], rel_emb.shape[0])
    mlp = _make_mlp(H, min(MLP_BLK, H))
    scores = []
    for k in range(NSPLIT):
        lo = k * H
        h, p = gather(ent_emb, rel_emb,
                      lax.slice(hidx, (lo,), (lo + H,)),
                      lax.slice(tidx, (lo,), (lo + H,)),
                      lax.slice(ridx, (lo,), (lo + H,)))
        scores.append(mlp(h, p, W1b, b1r, W2b, b2r, Wp, bpr))
    return jnp.concatenate(scores, axis=0)


# speedup vs baseline: 1.3597x; 1.1554x over previous
import functools

import jax
import jax.numpy as jnp
from jax import lax
from jax.experimental import pallas as pl
from jax.experimental.pallas import tpu as pltpu
from jax.experimental.pallas import tpu_sc as plsc

DIM = 128
CHUNK = 128
NSPLIT = 2
MLP_BLK = 4096


@functools.lru_cache(maxsize=None)
def _make_gather(B: int, H: int, lo: int, E: int, R: int):
    info = plsc.get_sparse_core_info()
    NC, NS = info.num_cores, info.num_subcores
    NW = NC * NS
    assert H % (8 * NW) == 0
    b_per_w = H // NW
    assert b_per_w % CHUNK == 0
    n_chunks = b_per_w // CHUNK

    mesh = plsc.VectorSubcoreMesh(core_axis_name="c", subcore_axis_name="s")

    @functools.partial(
        pl.kernel,
        mesh=mesh,
        out_type=(
            jax.ShapeDtypeStruct((H, DIM), jnp.float32),
            jax.ShapeDtypeStruct((H, DIM), jnp.float32),
        ),
        scratch_types=[
            pltpu.VMEM((CHUNK,), jnp.int32),
            pltpu.VMEM((CHUNK,), jnp.int32),
            pltpu.VMEM((CHUNK,), jnp.int32),
            pltpu.VMEM((CHUNK,), jnp.int32),
            pltpu.VMEM((CHUNK,), jnp.int32),
            pltpu.VMEM((CHUNK,), jnp.int32),
            pltpu.VMEM((CHUNK, DIM), jnp.float32),
            pltpu.VMEM((CHUNK, DIM), jnp.float32),
            pltpu.VMEM((CHUNK, DIM), jnp.float32),
            pltpu.VMEM((CHUNK, DIM), jnp.float32),
            pltpu.VMEM((CHUNK, DIM), jnp.float32),
            pltpu.VMEM((CHUNK, DIM), jnp.float32),
            pltpu.SemaphoreType.DMA,
            pltpu.SemaphoreType.DMA,
            pltpu.SemaphoreType.DMA,
            pltpu.SemaphoreType.DMA,
            pltpu.SemaphoreType.DMA,
            pltpu.SemaphoreType.DMA,
            pltpu.SemaphoreType.DMA,
            pltpu.SemaphoreType.DMA,
        ],
    )
    def gather_k(ent_hbm, rel_hbm, hidx_hbm, tidx_hbm, ridx_hbm,
                 h_out, p_out,
                 ih0, ih1, it0, it1, ir0, ir1,
                 bh0, bh1, bt0, bt1, br0, br1,
                 gsh0, gsh1, gst0, gst1, gsr0, gsr1, wsh, wsp):
        wid = lax.axis_index("s") * NC + lax.axis_index("c")
        base = wid * b_per_w
        ih = (ih0, ih1)
        it = (it0, it1)
        ir = (ir0, ir1)
        bh = (bh0, bh1)
        bt = (bt0, bt1)
        br = (br0, br1)
        gsh = (gsh0, gsh1)
        gst = (gst0, gst1)
        gsr = (gsr0, gsr1)

        def mul_into(tb, rb):
            def body(row, _):
                for g in range(DIM // 16):
                    sl = pl.ds(g * 16, 16)
                    tb[row, sl] = tb[row, sl] * rb[row, sl]
                return 0
            lax.fori_loop(0, CHUNK, body, 0)

        def start_unit(c):
            b = c & 1
            off = base + c * CHUNK
            pltpu.sync_copy(hidx_hbm.at[pl.ds(lo + off, CHUNK)], ih[b])
            pltpu.sync_copy(tidx_hbm.at[pl.ds(lo + off, CHUNK)], it[b])
            pltpu.sync_copy(ridx_hbm.at[pl.ds(lo + off, CHUNK)], ir[b])
            return (pltpu.async_copy(ent_hbm.at[ih[b]], bh[b], gsh[b]),
                    pltpu.async_copy(ent_hbm.at[it[b]], bt[b], gst[b]),
                    pltpu.async_copy(rel_hbm.at[ir[b]], br[b], gsr[b]))

        def finish_unit(c, handles):
            b = c & 1
            off = base + c * CHUNK
            hh, ht, hr = handles
            hh.wait()
            wh = pltpu.async_copy(bh[b], h_out.at[pl.ds(off, CHUNK)], wsh)
            ht.wait()
            hr.wait()
            mul_into(bt[b], br[b])
            wp = pltpu.async_copy(bt[b], p_out.at[pl.ds(off, CHUNK)], wsp)
            return wh, wp

        pend = [None] * n_chunks
        writes = [None] * n_chunks
        for c in range(n_chunks):
            if c >= 2:
                for w in writes[c - 2]:
                    w.wait()
            pend[c] = start_unit(c)
            if c >= 1:
                writes[c - 1] = finish_unit(c - 1, pend[c - 1])
        writes[n_chunks - 1] = finish_unit(n_chunks - 1, pend[n_chunks - 1])
        if n_chunks >= 2:
            for w in writes[n_chunks - 2]:
                w.wait()
        for w in writes[n_chunks - 1]:
            w.wait()

    return gather_k


def _mlp_body(h_ref, p_ref, W1_ref, b1_ref, W2_ref, b2_ref,
              Wp_ref, bp_ref, out_ref):
    x1 = h_ref[...]
    x2 = p_ref[...]
    ss = (jnp.sum(x1 * x1, axis=1, keepdims=True)
          + jnp.sum(x2 * x2, axis=1, keepdims=True))
    inv = 1.0 / jnp.maximum(jnp.sqrt(ss), 1e-12)
    x = jnp.concatenate([x1 * inv, x2 * inv], axis=1).astype(jnp.bfloat16)
    y = lax.dot_general(x, W1_ref[...], (((1,), (0,)), ((), ())),
                        preferred_element_type=jnp.float32) + b1_ref[...]
    y = jnp.maximum(y, 0.0).astype(jnp.bfloat16)
    y = lax.dot_general(y, W2_ref[...], (((1,), (0,)), ((), ())),
                        preferred_element_type=jnp.float32) + b2_ref[...]
    y = jnp.maximum(y, 0.0)
    s = jnp.sum(y * Wp_ref[...], axis=1, keepdims=True) + bp_ref[...]
    out_ref[...] = jax.nn.sigmoid(s)


@functools.lru_cache(maxsize=None)
def _make_mlp(H: int, blk: int):
    grid = (H // blk,)
    full = lambda i: (0, 0)
    return pl.pallas_call(
        _mlp_body,
        grid=grid,
        in_specs=[
            pl.BlockSpec((blk, DIM), lambda i: (i, 0)),
            pl.BlockSpec((blk, DIM), lambda i: (i, 0)),
            pl.BlockSpec((2 * DIM, 512), full),
            pl.BlockSpec((1, 512), full),
            pl.BlockSpec((512, 256), full),
            pl.BlockSpec((1, 256), full),
            pl.BlockSpec((1, 256), full),
            pl.BlockSpec((1, 1), full),
        ],
        out_specs=pl.BlockSpec((blk, 1), lambda i: (i, 0)),
        out_shape=jax.ShapeDtypeStruct((H, 1), jnp.float32),
    )


def kernel(data, eval, cf_train, ent_emb, rel_emb, W1, b1, W2, b2, Wp, bp):
    B = data.shape[0]
    hidx = data[:, 0]
    tidx = data[:, 1]
    ridx = data[:, 2]
    W1b = W1.T.astype(jnp.bfloat16)
    W2b = W2.T.astype(jnp.bfloat16)
    b1r = b1.reshape(1, -1)
    b2r = b2.reshape(1, -1)
    bpr = bp.reshape(1, 1)
    H = B // NSPLIT
    mlp = _make_mlp(H, min(MLP_BLK, H))
    scores = []
    for k in range(NSPLIT):
        gather = _make_gather(B, H, k * H, ent_emb.shape[0],
                              rel_emb.shape[0])
        h, p = gather(ent_emb, rel_emb, hidx, tidx, ridx)
        scores.append(mlp(h, p, W1b, b1r, W2b, b2r, Wp, bpr))
    return jnp.concatenate(scores, axis=0)
